# baseline (speedup 1.0000x reference)
import jax
import jax.numpy as jnp
from jax.experimental import pallas as pl

BLOCK_T = 512


def _router_kernel(x_ref, w_ref, o_ref):
    x = x_ref[...]
    w = w_ref[...]
    logits = jnp.dot(x, w, preferred_element_type=jnp.float32)
    m = jnp.max(logits, axis=-1, keepdims=True)
    e = jnp.exp(logits - m)
    o_ref[...] = e / jnp.sum(e, axis=-1, keepdims=True)


def kernel(states, W):
    T, D = states.shape
    E = W.shape[0]
    wt = W.T
    return pl.pallas_call(
        _router_kernel,
        grid=(T // BLOCK_T,),
        in_specs=[
            pl.BlockSpec((BLOCK_T, D), lambda i: (i, 0)),
            pl.BlockSpec((D, E), lambda i: (0, 0)),
        ],
        out_specs=pl.BlockSpec((BLOCK_T, E), lambda i: (i, 0)),
        out_shape=jax.ShapeDtypeStruct((T, E), jnp.float32),
    )(states, wt)


# speedup vs baseline: 1.0173x; 1.0173x over previous
import jax
import jax.numpy as jnp
from jax.experimental import pallas as pl

BLOCK_T = 1024


def _router_kernel(x_ref, w_ref, o_ref):
    x = x_ref[...]
    w = w_ref[...]
    logits = jnp.dot(x, w, preferred_element_type=jnp.float32)
    m = jnp.max(logits, axis=-1, keepdims=True)
    e = jnp.exp(logits - m)
    o_ref[...] = e / jnp.sum(e, axis=-1, keepdims=True)


def kernel(states, W):
    T, D = states.shape
    E = W.shape[0]
    wt = W.T
    return pl.pallas_call(
        _router_kernel,
        grid=(T // BLOCK_T,),
        in_specs=[
            pl.BlockSpec((BLOCK_T, D), lambda i: (i, 0)),
            pl.BlockSpec((D, E), lambda i: (0, 0)),
        ],
        out_specs=pl.BlockSpec((BLOCK_T, E), lambda i: (i, 0)),
        out_shape=jax.ShapeDtypeStruct((T, E), jnp.float32),
    )(states, wt)


# speedup vs baseline: 1.0188x; 1.0014x over previous
import jax
import jax.numpy as jnp
from jax.experimental import pallas as pl

BLOCK_T = 1024


def _router_kernel(x_ref, w_ref, o_ref):
    x = x_ref[...].astype(jnp.bfloat16)
    w = w_ref[...].astype(jnp.bfloat16)
    logits = jnp.dot(x, w, preferred_element_type=jnp.float32)
    m = jnp.max(logits, axis=-1, keepdims=True)
    e = jnp.exp(logits - m)
    o_ref[...] = e / jnp.sum(e, axis=-1, keepdims=True)


def kernel(states, W):
    T, D = states.shape
    E = W.shape[0]
    wt = W.T
    return pl.pallas_call(
        _router_kernel,
        grid=(T // BLOCK_T,),
        in_specs=[
            pl.BlockSpec((BLOCK_T, D), lambda i: (i, 0)),
            pl.BlockSpec((D, E), lambda i: (0, 0)),
        ],
        out_specs=pl.BlockSpec((BLOCK_T, E), lambda i: (i, 0)),
        out_shape=jax.ShapeDtypeStruct((T, E), jnp.float32),
    )(states, wt)


# speedup vs baseline: 1.0197x; 1.0009x over previous
import jax
import jax.numpy as jnp
from jax.experimental import pallas as pl
from jax.experimental.pallas import tpu as pltpu

BLOCK_T = 1024


def _router_kernel(x_ref, w_ref, o_ref):
    x = x_ref[...].astype(jnp.bfloat16)
    w = w_ref[...].astype(jnp.bfloat16)
    logits = jnp.dot(x, w, preferred_element_type=jnp.float32)
    m = jnp.max(logits, axis=-1, keepdims=True)
    e = jnp.exp(logits - m)
    o_ref[...] = e / jnp.sum(e, axis=-1, keepdims=True)


def kernel(states, W):
    T, D = states.shape
    E = W.shape[0]
    wt = W.T
    return pl.pallas_call(
        _router_kernel,
        grid=(T // BLOCK_T,),
        in_specs=[
            pl.BlockSpec((BLOCK_T, D), lambda i: (i, 0)),
            pl.BlockSpec((D, E), lambda i: (0, 0)),
        ],
        out_specs=pl.BlockSpec((BLOCK_T, E), lambda i: (i, 0)),
        out_shape=jax.ShapeDtypeStruct((T, E), jnp.float32),
        compiler_params=pltpu.CompilerParams(
            vmem_limit_bytes=100 * 1024 * 1024,
        ),
    )(states, wt)
